# baseline (speedup 1.0000x reference)
import functools

import jax
import jax.numpy as jnp
from jax import lax
from jax.experimental import pallas as pl
from jax.experimental.pallas import tpu as pltpu
from jax.experimental.pallas import tpu_sc as plsc

_NUM_SEGMENTS = 2
_EMBED_DIM = 1024
_BATCH = 4
_SEQ = 8192
_TOKENS = _BATCH * _SEQ
_NW = 32
_TOK_PER_W = _TOKENS // _NW
_CHUNK = 64
_NCHUNK = _TOK_PER_W // _CHUNK

_mesh = plsc.VectorSubcoreMesh(core_axis_name="c", subcore_axis_name="s")


@functools.partial(
    pl.kernel,
    mesh=_mesh,
    out_type=jax.ShapeDtypeStruct((_TOKENS, _EMBED_DIM), jnp.float32),
    scratch_types=[
        pltpu.VMEM((_TOK_PER_W,), jnp.int32),
        pltpu.VMEM((_CHUNK, _EMBED_DIM), jnp.float32),
        pltpu.SemaphoreType.DMA,
    ],
)
def _segment_gather(idx_hbm, table_hbm, out_hbm, idx_v, rows_v, sem):
    wid = lax.axis_index("s") * 2 + lax.axis_index("c")
    base = wid * _TOK_PER_W
    pltpu.sync_copy(idx_hbm.at[pl.ds(base, _TOK_PER_W)], idx_v)
    for i in range(_NCHUNK):
        ichunk = idx_v.at[pl.ds(i * _CHUNK, _CHUNK)]
        pltpu.async_copy(table_hbm.at[ichunk], rows_v, sem).wait()
        pltpu.sync_copy(rows_v, out_hbm.at[pl.ds(base + i * _CHUNK, _CHUNK)])


def kernel(inputs, segment_embed_weights):
    idx = inputs.astype(jnp.int32).reshape(_TOKENS)
    out = _segment_gather(idx, segment_embed_weights)
    return (out.reshape(_BATCH, _SEQ, _EMBED_DIM), segment_embed_weights)
